# Initial kernel scaffold; baseline (speedup 1.0000x reference)
import jax
import jax.numpy as jnp
from jax.experimental import pallas as pl


def kernel(x, embedding):
    raise NotImplementedError("write your pallas kernel here")


# speedup vs baseline: 1.1828x; 1.1828x over previous
import functools

import jax
import jax.numpy as jnp
from jax import lax
from jax.experimental import pallas as pl
from jax.experimental.pallas import tpu as pltpu
from jax.experimental.pallas import tpu_sc as plsc

KC = 1024
DC = 64
N = 32 * 576
M_BLK = 1152
GRID = N // M_BLK
LOSS_SCALE = 1.25 / (N * DC)


def _assign_body(x_ref, et_ref, idx_ref, loss_ref):
    i = pl.program_id(0)
    xb = x_ref[...]
    et = et_ref[...]
    e2 = jnp.sum(et * et, axis=0)
    x2 = jnp.sum(xb * xb, axis=1)
    dot = jnp.dot(xb, et, preferred_element_type=jnp.float32)
    s = (e2[None, :] + x2[:, None]) - 2.0 * dot
    min_s = jnp.min(s, axis=1)
    ks = lax.broadcasted_iota(jnp.int32, s.shape, 1)
    idx = jnp.min(jnp.where(s == min_s[:, None], ks, KC), axis=1)
    idx_ref[...] = idx.reshape(1, 1, M_BLK)
    part = jnp.sum(min_s)

    @pl.when(i == 0)
    def _():
        loss_ref[0, 0] = part

    @pl.when(i > 0)
    def _():
        loss_ref[0, 0] += part

    @pl.when(i == GRID - 1)
    def _():
        loss_ref[0, 0] *= LOSS_SCALE


_assign = pl.pallas_call(
    _assign_body,
    grid=(GRID,),
    in_specs=[
        pl.BlockSpec((M_BLK, DC), lambda i: (i, 0)),
        pl.BlockSpec((DC, KC), lambda i: (0, 0)),
    ],
    out_specs=[
        pl.BlockSpec((1, 1, M_BLK), lambda i: (i, 0, 0)),
        pl.BlockSpec((1, 1), lambda i: (0, 0), memory_space=pltpu.SMEM),
    ],
    out_shape=[
        jax.ShapeDtypeStruct((GRID, 1, M_BLK), jnp.int32),
        jax.ShapeDtypeStruct((1, 1), jnp.float32),
    ],
    compiler_params=pltpu.CompilerParams(dimension_semantics=("arbitrary",)),
)

_NW = 32
_ROWS_W = N // _NW
_CH = 96
_NCH = _ROWS_W // _CH


@functools.cache
def _make_gather():
    mesh = plsc.VectorSubcoreMesh(core_axis_name="c", subcore_axis_name="s")

    @functools.partial(
        pl.kernel,
        mesh=mesh,
        compiler_params=pltpu.CompilerParams(use_tc_tiling_on_sc=False),
        out_type=jax.ShapeDtypeStruct((N, DC), jnp.float32),
        scratch_types=[
            pltpu.VMEM((_NCH, _CH), jnp.int32),
            pltpu.VMEM((_ROWS_W, DC), jnp.float32),
            pltpu.SemaphoreType.DMA,
        ],
    )
    def gather(table_hbm, idx_hbm, out_hbm, idx_v, rows_v, sem):
        wid = lax.axis_index("s") * 2 + lax.axis_index("c")
        pltpu.sync_copy(idx_hbm.at[wid], idx_v)
        copies = [
            pltpu.async_copy(
                table_hbm.at[idx_v.at[j]], rows_v.at[pl.ds(j * _CH, _CH)], sem
            )
            for j in range(_NCH)
        ]
        for c in copies:
            c.wait()
        pltpu.sync_copy(rows_v, out_hbm.at[pl.ds(wid * _ROWS_W, _ROWS_W)])

    return gather


def kernel(x, embedding):
    x_flat = x.reshape(N, DC)
    idx, loss = _assign(x_flat, embedding.T)
    quant = _make_gather()(embedding, idx.reshape(_NW, _NCH, _CH))
    return quant.reshape(x.shape), loss[0, 0]
